# Initial kernel scaffold; baseline (speedup 1.0000x reference)
import jax
import jax.numpy as jnp
from jax.experimental import pallas as pl


def kernel(x, edge_index, W1_l, W1_r, b1, W2_l, W2_r, b2):
    raise NotImplementedError("write your pallas kernel here")


# speedup vs baseline: 3.0570x; 3.0570x over previous
import functools

import jax
import jax.numpy as jnp
from jax import lax
from jax.experimental import pallas as pl
from jax.experimental.pallas import tpu as pltpu
from jax.experimental.pallas import tpu_sc as plsc

N = 10000
E = 320000
D_IN = 128
D_HID = 64
D_OUT = 128

NC = 2
NS = 16
NW = NC * NS
CHUNK = 128
EPW = 10240
E_PAD = NW * EPW
NCHUNK = EPW // CHUNK
WT = 128
N_T = 10008

ROWS_PER_SUB = 624
TAIL_BASE = ROWS_PER_SUB * NS
TAIL_ROWS = N_T - TAIL_BASE


def _make_sc_agg():
    mesh = plsc.VectorSubcoreMesh(core_axis_name="c", subcore_axis_name="s",
                                  num_cores=NC, num_subcores=NS)

    @functools.partial(
        pl.kernel,
        out_type=jax.ShapeDtypeStruct((NC, N_T, WT), jnp.float32),
        mesh=mesh,
        scratch_types=[
            pltpu.VMEM((NCHUNK, CHUNK), jnp.int32),
            pltpu.VMEM((NCHUNK, CHUNK), jnp.int32),
            pltpu.VMEM((CHUNK, WT), jnp.float32),
            pltpu.VMEM_SHARED((N_T, WT), jnp.float32),
            pltpu.SemaphoreType.DMA,
        ],
    )
    def sc_agg(table_hbm, src_hbm, dst_hbm, zeros_hbm, out_hbm,
               src_v, dst_v, msg, acc_sh, sem):
        cid = lax.axis_index("c")
        sid = lax.axis_index("s")
        wid = sid * NC + cid

        pltpu.sync_copy(zeros_hbm.at[pl.ds(sid * ROWS_PER_SUB, ROWS_PER_SUB)],
                        acc_sh.at[pl.ds(sid * ROWS_PER_SUB, ROWS_PER_SUB)])

        @pl.when(sid == 0)
        def _():
            pltpu.sync_copy(zeros_hbm.at[pl.ds(TAIL_BASE, TAIL_ROWS)],
                            acc_sh.at[pl.ds(TAIL_BASE, TAIL_ROWS)])

        pltpu.sync_copy(src_hbm.at[wid], src_v)
        pltpu.sync_copy(dst_hbm.at[wid], dst_v)
        plsc.subcore_barrier()

        def body(j, carry):
            pltpu.async_copy(table_hbm.at[src_v.at[j]], msg, sem).wait()
            pltpu.sync_copy(msg, acc_sh.at[dst_v.at[j]], add=True)
            return carry

        lax.fori_loop(0, NCHUNK, body, 0)

        plsc.subcore_barrier()
        pltpu.sync_copy(acc_sh.at[pl.ds(sid * ROWS_PER_SUB, ROWS_PER_SUB)],
                        out_hbm.at[cid, pl.ds(sid * ROWS_PER_SUB, ROWS_PER_SUB)])

        @pl.when(sid == 0)
        def _():
            pltpu.sync_copy(acc_sh.at[pl.ds(TAIL_BASE, TAIL_ROWS)],
                            out_hbm.at[cid, pl.ds(TAIL_BASE, TAIL_ROWS)])

    return sc_agg


_sc_agg_cache = []


def _sc_agg():
    if not _sc_agg_cache:
        _sc_agg_cache.append(_make_sc_agg())
    return _sc_agg_cache[0]


def _proj1_body(x_ref, w_ref, o_ref):
    y = jnp.dot(x_ref[...], w_ref[...], preferred_element_type=jnp.float32)
    ones = jnp.ones((N, 1), jnp.float32)
    pad = jnp.zeros((N, WT - D_HID - 1), jnp.float32)
    rowpad = jnp.zeros((N_T - N, WT), jnp.float32)
    o_ref[...] = jnp.concatenate(
        [jnp.concatenate([y, ones, pad], axis=1), rowpad], axis=0)


def _mid_body(acc_ref, x_ref, w_ref, b_ref, o_ref):
    a = (acc_ref[0] + acc_ref[1])[:N]
    deg = a[:, D_HID:D_HID + 1]
    mean = a[:, :D_HID] / jnp.clip(deg, 1.0, None)
    xw = jnp.dot(x_ref[...], w_ref[...], preferred_element_type=jnp.float32)
    h = jax.nn.relu(mean + xw + b_ref[...])
    ones = jnp.ones((N, 1), jnp.float32)
    pad = jnp.zeros((N, WT - D_HID - 1), jnp.float32)
    rowpad = jnp.zeros((N_T - N, WT), jnp.float32)
    o_ref[...] = jnp.concatenate(
        [jnp.concatenate([h, ones, pad], axis=1), rowpad], axis=0)


def _final_body(acc2_ref, h_ref, wl_ref, wr_ref, b_ref, o_ref):
    a = (acc2_ref[0] + acc2_ref[1])[:N]
    deg = a[:, D_HID:D_HID + 1]
    mean = a[:, :D_HID] / jnp.clip(deg, 1.0, None)
    mm = jnp.dot(mean, wl_ref[...], preferred_element_type=jnp.float32)
    hw = jnp.dot(h_ref[:N, :D_HID], wr_ref[...],
                 preferred_element_type=jnp.float32)
    o_ref[...] = jax.nn.relu(mm + hw + b_ref[...])


def kernel(x, edge_index, W1_l, W1_r, b1, W2_l, W2_r, b2):
    ei = edge_index.astype(jnp.int32)
    pad = jnp.full((2, E_PAD - E), N, jnp.int32)
    ei = jnp.concatenate([ei, pad], axis=1)
    src = ei[0].reshape(NW, NCHUNK, CHUNK)
    dst = ei[1].reshape(NW, NCHUNK, CHUNK)
    zeros = jnp.zeros((N_T, WT), jnp.float32)

    y1t = pl.pallas_call(
        _proj1_body,
        out_shape=jax.ShapeDtypeStruct((N_T, WT), jnp.float32),
    )(x, W1_l)

    acc1 = _sc_agg()(y1t, src, dst, zeros)

    ht = pl.pallas_call(
        _mid_body,
        out_shape=jax.ShapeDtypeStruct((N_T, WT), jnp.float32),
    )(acc1, x, W1_r, b1.reshape(1, D_HID))

    acc2 = _sc_agg()(ht, src, dst, zeros)

    out = pl.pallas_call(
        _final_body,
        out_shape=jax.ShapeDtypeStruct((N, D_OUT), jnp.float32),
    )(acc2, ht, W2_l, W2_r, b2.reshape(1, D_OUT))

    return out
